# Initial kernel scaffold; baseline (speedup 1.0000x reference)
import jax
import jax.numpy as jnp
from jax.experimental import pallas as pl


def kernel(input_ids, token_type_ids, word_emb, pos_emb, type_emb, ln_gamma, ln_beta):
    raise NotImplementedError("write your pallas kernel here")


# speedup vs baseline: 10.8698x; 10.8698x over previous
import functools

import jax
import jax.numpy as jnp
from jax import lax
from jax.experimental import pallas as pl
from jax.experimental.pallas import tpu as pltpu
from jax.experimental.pallas import tpu_sc as plsc

HIDDEN = 128
SEQ = 200
BATCH = 1024
EPS = 1e-12

NC = 2
NS = 16
NW = NC * NS
TOKENS = BATCH * SEQ
TOK_PER_W = TOKENS // NW
CHUNK = 128
CHUNKS_PER_W = TOK_PER_W // CHUNK


def _sc_gather_body(word_hbm, ids_hbm, out_hbm, idx_v, buf0, buf1, sem0, sem1):
    wid = lax.axis_index("s") * NC + lax.axis_index("c")
    base_tok = wid * TOK_PER_W
    pltpu.sync_copy(ids_hbm.at[pl.ds(base_tok, TOK_PER_W)], idx_v)
    bufs = (buf0, buf1)
    sems = (sem0, sem1)
    handles = [None, None]
    handles[0] = pltpu.async_copy(
        word_hbm.at[idx_v.at[pl.ds(0, CHUNK)]], bufs[0], sems[0])
    for j in range(CHUNKS_PER_W):
        nxt = j + 1
        if nxt < CHUNKS_PER_W:
            handles[nxt % 2] = pltpu.async_copy(
                word_hbm.at[idx_v.at[pl.ds(nxt * CHUNK, CHUNK)]],
                bufs[nxt % 2], sems[nxt % 2])
        handles[j % 2].wait()
        pltpu.sync_copy(bufs[j % 2],
                        out_hbm.at[pl.ds(base_tok + j * CHUNK, CHUNK)])


@functools.cache
def _get_sc_gather():
    return pl.kernel(
        _sc_gather_body,
        out_type=jax.ShapeDtypeStruct((TOKENS, HIDDEN), jnp.float32),
        mesh=plsc.VectorSubcoreMesh(core_axis_name="c", subcore_axis_name="s"),
        scratch_types=[
            pltpu.VMEM((TOK_PER_W,), jnp.int32),
            pltpu.VMEM((CHUNK, HIDDEN), jnp.float32),
            pltpu.VMEM((CHUNK, HIDDEN), jnp.float32),
            pltpu.SemaphoreType.DMA,
            pltpu.SemaphoreType.DMA,
        ],
    )


BB = 16


def _tc_post_body(g_ref, tt_ref, pos_ref, type_ref, gamma_ref, beta_ref, o_ref):
    x = g_ref[...]
    pos = pos_ref[0:SEQ, :][None, :, :]
    t0 = type_ref[0:1, :][None, :, :]
    t1 = type_ref[1:2, :][None, :, :]
    tt = tt_ref[...].astype(jnp.float32)[:, :, None]
    x = x + pos + t0 + tt * (t1 - t0)
    mean = jnp.mean(x, axis=-1, keepdims=True)
    xc = x - mean
    var = jnp.mean(xc * xc, axis=-1, keepdims=True)
    inv = lax.rsqrt(var + EPS)
    gamma = gamma_ref[...][:, None, :]
    beta = beta_ref[...][:, None, :]
    o_ref[...] = xc * inv * gamma + beta


def _tc_post(gathered3, token_type_ids, pos_emb, type_emb, gamma2, beta2):
    return pl.pallas_call(
        _tc_post_body,
        grid=(BATCH // BB,),
        in_specs=[
            pl.BlockSpec((BB, SEQ, HIDDEN), lambda i: (i, 0, 0)),
            pl.BlockSpec((BB, SEQ), lambda i: (i, 0)),
            pl.BlockSpec((512, HIDDEN), lambda i: (0, 0)),
            pl.BlockSpec((2, HIDDEN), lambda i: (0, 0)),
            pl.BlockSpec((1, HIDDEN), lambda i: (0, 0)),
            pl.BlockSpec((1, HIDDEN), lambda i: (0, 0)),
        ],
        out_specs=pl.BlockSpec((BB, SEQ, HIDDEN), lambda i: (i, 0, 0)),
        out_shape=jax.ShapeDtypeStruct((BATCH, SEQ, HIDDEN), jnp.float32),
    )(gathered3, token_type_ids, pos_emb, type_emb, gamma2, beta2)


def kernel(input_ids, token_type_ids, word_emb, pos_emb, type_emb, ln_gamma, ln_beta):
    ids = input_ids.astype(jnp.int32).reshape(TOKENS)
    gathered = _get_sc_gather()(word_emb, ids)
    return _tc_post(
        gathered.reshape(BATCH, SEQ, HIDDEN),
        token_type_ids.astype(jnp.int32),
        pos_emb,
        type_emb,
        ln_gamma.reshape(1, HIDDEN),
        ln_beta.reshape(1, HIDDEN),
    )
